# Initial kernel scaffold; baseline (speedup 1.0000x reference)
import jax
import jax.numpy as jnp
from jax.experimental import pallas as pl


def kernel(query, support, table, proj1_w, proj1_b, proj2_w, proj2_b, ln_a, ln_b, w_ih, w_hh, b_ih, b_hh):
    raise NotImplementedError("write your pallas kernel here")


# speedup vs baseline: 3.3150x; 3.3150x over previous
import functools

import jax
import jax.numpy as jnp
from jax import lax
from jax.experimental import pallas as pl
from jax.experimental.pallas import tpu as pltpu
from jax.experimental.pallas import tpu_sc as plsc

_EMBED_DIM = 128
_D_MODEL = 2 * _EMBED_DIM
_HIDDEN = 2 * _D_MODEL
_STEPS = 4
_SUP_PAD = 8

_NC = 2
_NS = 16
_NW = _NC * _NS


def _sc_gather(table, idx_all):
    n_rows = idx_all.shape[0]
    b_per_w = n_rows // _NW
    mesh = plsc.VectorSubcoreMesh(core_axis_name="c", subcore_axis_name="s")

    @functools.partial(
        pl.kernel,
        mesh=mesh,
        out_type=jax.ShapeDtypeStruct((n_rows, _EMBED_DIM), jnp.float32),
        scratch_types=[
            pltpu.VMEM((b_per_w,), jnp.int32),
            pltpu.VMEM((b_per_w, _EMBED_DIM), jnp.float32),
            pltpu.SemaphoreType.DMA,
        ],
    )
    def gather_kernel(table_hbm, idx_hbm, out_hbm, idx_v, rows_v, sem):
        wid = lax.axis_index("s") * _NC + lax.axis_index("c")
        base = wid * b_per_w
        pltpu.sync_copy(idx_hbm.at[pl.ds(base, b_per_w)], idx_v)
        pltpu.async_copy(table_hbm.at[idx_v], rows_v, sem).wait()
        pltpu.sync_copy(rows_v, out_hbm.at[pl.ds(base, b_per_w)])

    return gather_kernel(table, idx_all)


def _sigmoid(x):
    return 1.0 / (1.0 + jnp.exp(-x))


def _matcher_body(few, qb_ref, sp_ref, p1w_ref, p1b_ref, p2w_ref, p2b_ref,
                  lna_ref, lnb_ref, wih_ref, whh_ref, bih_ref, bhh_ref,
                  out_ref):
    f32 = jnp.float32
    dims = (((1,), (1,)), ((), ()))

    s = sp_ref[...]
    h1 = lax.dot_general(s, p1w_ref[...], dims, preferred_element_type=f32)
    h1 = jnp.maximum(h1 + p1b_ref[...], 0.0)
    h2 = lax.dot_general(h1, p2w_ref[...], dims, preferred_element_type=f32)
    z = h2 + p2b_ref[...] + s
    mu = jnp.mean(z, axis=1, keepdims=True)
    zc = z - mu
    var = jnp.sum(zc * zc, axis=1, keepdims=True) / (_D_MODEL - 1)
    sg = zc / (jnp.sqrt(var) + 1e-6) * lna_ref[...] + lnb_ref[...]
    row = lax.broadcasted_iota(jnp.int32, (_SUP_PAD, 1), 0)
    sg = jnp.where(row < few, sg, 0.0)

    whh = whh_ref[...]
    whh_h = whh[:, :_D_MODEL]
    whh_r = whh[:, _D_MODEL:]
    s_r = lax.dot_general(sg, whh_r, dims, preferred_element_type=f32)

    qb = qb_ref[...]
    g0 = lax.dot_general(qb, wih_ref[...], dims, preferred_element_type=f32)
    g0 = g0 + bih_ref[...] + bhh_ref[...]
    qh = lax.dot_general(qb, whh_h, dims, preferred_element_type=f32)
    qs = lax.dot_general(qb, sg, dims, preferred_element_type=f32)

    col = lax.broadcasted_iota(jnp.int32, (1, _SUP_PAD), 1)
    gates = g0
    c = None
    for t in range(_STEPS):
        gi = _sigmoid(gates[:, :_HIDDEN])
        gf = _sigmoid(gates[:, _HIDDEN:2 * _HIDDEN])
        gg = jnp.tanh(gates[:, 2 * _HIDDEN:3 * _HIDDEN])
        go = _sigmoid(gates[:, 3 * _HIDDEN:])
        c = gi * gg if c is None else gf * c + gi * gg
        hc = go * jnp.tanh(c)
        hch = hc[:, :_D_MODEL]
        logits = qs + lax.dot_general(hch, sg, dims, preferred_element_type=f32)
        if t == _STEPS - 1:
            out_ref[...] = logits
        else:
            lm = jnp.where(col < few, logits, -1e30)
            m = jnp.max(lm, axis=1, keepdims=True)
            e = jnp.exp(lm - m)
            attn = e / jnp.sum(e, axis=1, keepdims=True)
            gates = (g0 + qh
                     + lax.dot_general(hch, whh_h, dims,
                                       preferred_element_type=f32)
                     + jnp.dot(attn, s_r, preferred_element_type=f32))


def _matcher_call(q, s_p, proj1_w, proj1_b, proj2_w, proj2_b, ln_a, ln_b,
                  w_ih, w_hh, b_ih, b_hh, few, blk):
    batch = q.shape[0]
    nb = batch // blk
    whole = lambda shape: pl.BlockSpec(shape, lambda i: (0, 0))
    return pl.pallas_call(
        functools.partial(_matcher_body, few),
        grid=(nb,),
        in_specs=[
            pl.BlockSpec((blk, _D_MODEL), lambda i: (i, 0)),
            whole((_SUP_PAD, _D_MODEL)),
            whole(proj1_w.shape),
            whole((1, proj1_b.shape[0])),
            whole(proj2_w.shape),
            whole((1, proj2_b.shape[0])),
            whole((1, ln_a.shape[0])),
            whole((1, ln_b.shape[0])),
            whole(w_ih.shape),
            whole(w_hh.shape),
            whole((1, b_ih.shape[0])),
            whole((1, b_hh.shape[0])),
        ],
        out_specs=pl.BlockSpec((blk, _SUP_PAD), lambda i: (i, 0)),
        out_shape=jax.ShapeDtypeStruct((batch, _SUP_PAD), jnp.float32),
        compiler_params=pltpu.CompilerParams(
            dimension_semantics=("arbitrary",)),
    )(q, s_p, proj1_w, proj1_b.reshape(1, -1), proj2_w,
      proj2_b.reshape(1, -1), ln_a.reshape(1, -1), ln_b.reshape(1, -1),
      w_ih, w_hh, b_ih.reshape(1, -1), b_hh.reshape(1, -1))


def kernel(query, support, table, proj1_w, proj1_b, proj2_w, proj2_b,
           ln_a, ln_b, w_ih, w_hh, b_ih, b_hh):
    batch = query.shape[0]
    few = support.shape[0]

    qi = query.reshape(-1).astype(jnp.int32)
    si = support.reshape(-1).astype(jnp.int32)
    n_idx = qi.shape[0] + si.shape[0]
    align = 8 * _NW
    n_pad = (-n_idx) % align
    zero_row = table.shape[0] - 1
    idx_all = jnp.concatenate(
        [qi, si, jnp.full((n_pad,), zero_row, jnp.int32)])
    rows = _sc_gather(table, idx_all)

    q = rows[:2 * batch].reshape(batch, _D_MODEL)
    s = rows[2 * batch:2 * batch + 2 * few].reshape(few, _D_MODEL)
    s_p = jnp.pad(s, ((0, _SUP_PAD - few), (0, 0)))

    out_p = _matcher_call(q, s_p, proj1_w, proj1_b, proj2_w, proj2_b,
                          ln_a, ln_b, w_ih, w_hh, b_ih, b_hh, few, blk=512)
    return out_p[:, :few]
